# Initial kernel scaffold; baseline (speedup 1.0000x reference)
import jax
import jax.numpy as jnp
from jax.experimental import pallas as pl


def kernel(pos_v, pos_u, neg_u, v_table, u_table):
    raise NotImplementedError("write your pallas kernel here")


# speedup vs baseline: 1.5739x; 1.5739x over previous
import functools

import jax
import jax.numpy as jnp
from jax import lax
from jax.experimental import pallas as pl
from jax.experimental.pallas import tpu as pltpu
from jax.experimental.pallas import tpu_sc as plsc

B = 16384
D = 64
K = 5
NC = 2
NS = 16
NW = NC * NS
BPW = B // NW
C = 128
NCH = BPW // C
L = 16


def _sc_body(pos_v_hbm, pos_u_hbm, negu_hbm, vtab_hbm, utab_hbm,
             spos_hbm, sneg_hbm,
             idxv, idxu, idxn, rv, ru, rn, sp, sn, sem):
    wid = lax.axis_index("s") * NC + lax.axis_index("c")
    base = wid * BPW

    for ch in range(NCH):
        cb = base + ch * C
        pltpu.sync_copy(pos_v_hbm.at[pl.ds(cb, C)], idxv)
        pltpu.sync_copy(pos_u_hbm.at[pl.ds(cb, C)], idxu)
        pltpu.sync_copy(negu_hbm.at[pl.ds(cb * K, C * K)], idxn)
        cpv = pltpu.async_copy(vtab_hbm.at[idxv], rv, sem)
        cpu = pltpu.async_copy(utab_hbm.at[idxu], ru, sem)
        cpn = pltpu.async_copy(utab_hbm.at[idxn], rn, sem)
        cpv.wait()
        cpu.wait()
        cpn.wait()

        def group_body(g, _, _ch=ch):
            bidx = lax.broadcasted_iota(jnp.int32, (L,), 0) + g * L
            nrow = bidx * K
            accp = jnp.zeros((L,), jnp.float32)
            accn = jnp.zeros((L,), jnp.float32)
            for d in range(D):
                dv = jnp.full((L,), d, jnp.int32)
                vv = plsc.load_gather(rv, [bidx, dv])
                uu = plsc.load_gather(ru, [bidx, dv])
                accp = accp + vv * uu
                nacc = plsc.load_gather(rn, [nrow, dv])
                for k in range(1, K):
                    nacc = nacc + plsc.load_gather(rn, [nrow + k, dv])
                accn = accn + nacc * vv
            sp[pl.ds(_ch * C + g * L, L)] = accp
            sn[pl.ds(_ch * C + g * L, L)] = accn
            return 0

        lax.fori_loop(0, C // L, group_body, 0)

    pltpu.sync_copy(sp, spos_hbm.at[pl.ds(base, BPW)])
    pltpu.sync_copy(sn, sneg_hbm.at[pl.ds(base, BPW)])


_sc_dots = functools.partial(
    pl.kernel,
    out_type=(jax.ShapeDtypeStruct((B,), jnp.float32),
              jax.ShapeDtypeStruct((B,), jnp.float32)),
    mesh=plsc.VectorSubcoreMesh(core_axis_name="c", subcore_axis_name="s"),
    scratch_types=[
        pltpu.VMEM((C,), jnp.int32),
        pltpu.VMEM((C,), jnp.int32),
        pltpu.VMEM((C * K,), jnp.int32),
        pltpu.VMEM((C, D), jnp.float32),
        pltpu.VMEM((C, D), jnp.float32),
        pltpu.VMEM((C * K, D), jnp.float32),
        pltpu.VMEM((BPW,), jnp.float32),
        pltpu.VMEM((BPW,), jnp.float32),
        pltpu.SemaphoreType.DMA,
    ],
    compiler_params=pltpu.CompilerParams(needs_layout_passes=False,
                                         use_tc_tiling_on_sc=False),
)(_sc_body)


def _tc_loss_body(sp_ref, sn_ref, out_ref):
    sp = sp_ref[...]
    sn = sn_ref[...]

    def logsig(x):
        return jnp.minimum(x, 0.0) - jnp.log1p(jnp.exp(-jnp.abs(x)))

    out_ref[0, 0] = -(jnp.sum(logsig(sp)) + jnp.sum(logsig(-sn)))


_tc_loss = pl.pallas_call(
    _tc_loss_body,
    out_shape=jax.ShapeDtypeStruct((1, 1), jnp.float32),
    out_specs=pl.BlockSpec(memory_space=pltpu.SMEM),
)


def kernel(pos_v, pos_u, neg_u, v_table, u_table):
    pos_v = pos_v.astype(jnp.int32)
    pos_u = pos_u.astype(jnp.int32)
    neg_flat = neg_u.astype(jnp.int32).reshape(-1)
    sp, sn = _sc_dots(pos_v, pos_u, neg_flat, v_table, u_table)
    loss = _tc_loss(sp.reshape(128, 128), sn.reshape(128, 128))
    return loss[0, 0]


# speedup vs baseline: 2.2859x; 1.4524x over previous
import functools

import jax
import jax.numpy as jnp
from jax import lax
from jax.experimental import pallas as pl
from jax.experimental.pallas import tpu as pltpu
from jax.experimental.pallas import tpu_sc as plsc

V = 1000000
B = 16384
D = 64
K = 5
R = K + 2
NC = 2
NS = 16
NW = NC * NS
BPW = B // NW
C = 64
NCH = BPW // C
L = 16


def _sc_body(pos_v_hbm, pos_u_hbm, negu_hbm, vtab_hbm, utab_hbm,
             spos_hbm, sneg_hbm,
             idxv, idxu, idxn, rows, sp, sn, sem):
    wid = lax.axis_index("s") * NC + lax.axis_index("c")
    base = wid * BPW

    def chunk_body(ch, _carry):
        cb = base + ch * C
        pltpu.sync_copy(pos_v_hbm.at[pl.ds(cb, C)], idxv)
        pltpu.sync_copy(pos_u_hbm.at[pl.ds(cb, C)], idxu)
        pltpu.sync_copy(negu_hbm.at[pl.ds(cb * K, C * K)], idxn)

        def issue(g, _):
            gb = g * L
            vecv = idxv[pl.ds(gb, L)]
            vecu = idxu[pl.ds(gb, L)]
            vecn = [idxn[pl.ds(gb * K + k * L, L)] for k in range(K)]
            for i in range(L):
                rb = (gb + i) * R
                pltpu.async_copy(vtab_hbm.at[pl.ds(vecv[i], 1)],
                                 rows.at[pl.ds(rb, 1)], sem)
                pltpu.async_copy(utab_hbm.at[pl.ds(vecu[i], 1)],
                                 rows.at[pl.ds(rb + 1, 1)], sem)
                ii = i * K
                for k in range(K):
                    jn = vecn[(ii + k) // L][(ii + k) % L]
                    pltpu.async_copy(utab_hbm.at[pl.ds(jn, 1)],
                                     rows.at[pl.ds(rb + 2 + k, 1)], sem)
            return 0

        lax.fori_loop(0, C // L, issue, 0)
        pltpu.make_async_copy(vtab_hbm.at[pl.ds(0, C * R)], rows, sem).wait()

        def group_body(g, _):
            bidx = lax.broadcasted_iota(jnp.int32, (L,), 0) + g * L
            rowb = bidx * R
            accp = jnp.zeros((L,), jnp.float32)
            accn = jnp.zeros((L,), jnp.float32)
            for d in range(D):
                dv = jnp.full((L,), d, jnp.int32)
                vv = plsc.load_gather(rows, [rowb, dv])
                uu = plsc.load_gather(rows, [rowb + 1, dv])
                accp = accp + vv * uu
                nacc = plsc.load_gather(rows, [rowb + 2, dv])
                for k in range(1, K):
                    nacc = nacc + plsc.load_gather(rows, [rowb + 2 + k, dv])
                accn = accn + nacc * vv
            sp[pl.ds(ch * C + g * L, L)] = accp
            sn[pl.ds(ch * C + g * L, L)] = accn
            return 0

        lax.fori_loop(0, C // L, group_body, 0)
        return 0

    lax.fori_loop(0, NCH, chunk_body, 0)

    pltpu.sync_copy(sp, spos_hbm.at[pl.ds(base, BPW)])
    pltpu.sync_copy(sn, sneg_hbm.at[pl.ds(base, BPW)])


_sc_dots = functools.partial(
    pl.kernel,
    out_type=(jax.ShapeDtypeStruct((B,), jnp.float32),
              jax.ShapeDtypeStruct((B,), jnp.float32)),
    mesh=plsc.VectorSubcoreMesh(core_axis_name="c", subcore_axis_name="s"),
    scratch_types=[
        pltpu.VMEM((C,), jnp.int32),
        pltpu.VMEM((C,), jnp.int32),
        pltpu.VMEM((C * K,), jnp.int32),
        pltpu.VMEM((C * R, D), jnp.float32),
        pltpu.VMEM((BPW,), jnp.float32),
        pltpu.VMEM((BPW,), jnp.float32),
        pltpu.SemaphoreType.DMA,
    ],
    compiler_params=pltpu.CompilerParams(needs_layout_passes=False),
)(_sc_body)


def _tc_loss_body(sp_ref, sn_ref, out_ref):
    sp = sp_ref[...]
    sn = sn_ref[...]

    def logsig(x):
        return jnp.minimum(x, 0.0) - jnp.log1p(jnp.exp(-jnp.abs(x)))

    out_ref[0, 0] = -(jnp.sum(logsig(sp)) + jnp.sum(logsig(-sn)))


_tc_loss = pl.pallas_call(
    _tc_loss_body,
    out_shape=jax.ShapeDtypeStruct((1, 1), jnp.float32),
    out_specs=pl.BlockSpec(memory_space=pltpu.SMEM),
)


def kernel(pos_v, pos_u, neg_u, v_table, u_table):
    pos_v = pos_v.astype(jnp.int32)
    pos_u = pos_u.astype(jnp.int32)
    neg_flat = neg_u.astype(jnp.int32).reshape(-1)
    sp, sn = _sc_dots(pos_v, pos_u, neg_flat, v_table, u_table)
    loss = _tc_loss(sp.reshape(128, 128), sn.reshape(128, 128))
    return loss[0, 0]


# speedup vs baseline: 2.5376x; 1.1101x over previous
import functools

import jax
import jax.numpy as jnp
from jax import lax
from jax.experimental import pallas as pl
from jax.experimental.pallas import tpu as pltpu
from jax.experimental.pallas import tpu_sc as plsc

V = 1000000
B = 16384
D = 64
K = 5
R = K + 2
NC = 2
NS = 16
NW = NC * NS
BPW = B // NW
C = 64
NCH = BPW // C
L = 16


def _sc_body(pos_v_hbm, pos_u_hbm, negu_hbm, vtab_hbm, utab_hbm,
             spos_hbm, sneg_hbm,
             idxv, idxu, idxn, rows, sp, sn, sem):
    wid = lax.axis_index("s") * NC + lax.axis_index("c")
    base = wid * BPW
    lane = lax.broadcasted_iota(jnp.int32, (L,), 0)

    def chunk_body(ch, _carry):
        cb = base + ch * C
        pltpu.sync_copy(pos_v_hbm.at[pl.ds(cb, C)], idxv)
        pltpu.sync_copy(pos_u_hbm.at[pl.ds(cb, C)], idxu)
        pltpu.sync_copy(negu_hbm.at[pl.ds(cb * K, C * K)], idxn)

        def issue(g, _):
            gb = g * L
            vecv = idxv[pl.ds(gb, L)]
            vecu = idxu[pl.ds(gb, L)]
            vecn = [idxn[pl.ds(gb * K + k * L, L)] for k in range(K)]
            for i in range(L):
                rb = (gb + i) * R
                pltpu.async_copy(vtab_hbm.at[pl.ds(vecv[i], 1)],
                                 rows.at[pl.ds(rb, 1)], sem)
                pltpu.async_copy(utab_hbm.at[pl.ds(vecu[i], 1)],
                                 rows.at[pl.ds(rb + 1, 1)], sem)
                ii = i * K
                for k in range(K):
                    jn = vecn[(ii + k) // L][(ii + k) % L]
                    pltpu.async_copy(utab_hbm.at[pl.ds(jn, 1)],
                                     rows.at[pl.ds(rb + 2 + k, 1)], sem)
            return 0

        lax.fori_loop(0, C // L, issue, 0)
        pltpu.make_async_copy(vtab_hbm.at[pl.ds(0, C * R)], rows, sem).wait()

        def compute_b(b, _):
            rb = b * R
            pv = jnp.zeros((L,), jnp.float32)
            nv = jnp.zeros((L,), jnp.float32)
            for j in range(D // L):
                s = pl.ds(j * L, L)
                vj = rows[rb, s]
                pv = pv + vj * rows[rb + 1, s]
                nsum = rows[rb + 2, s]
                for k in range(1, K):
                    nsum = nsum + rows[rb + 2 + k, s]
                nv = nv + vj * nsum
            sp[pl.ds((ch * C + b) * L, L)] = pv
            sn[pl.ds((ch * C + b) * L, L)] = nv
            return 0

        lax.fori_loop(0, C, compute_b, 0)
        return 0

    lax.fori_loop(0, NCH, chunk_body, 0)

    pltpu.sync_copy(sp, spos_hbm.at[pl.ds(base * L, BPW * L)])
    pltpu.sync_copy(sn, sneg_hbm.at[pl.ds(base * L, BPW * L)])


_sc_dots = functools.partial(
    pl.kernel,
    out_type=(jax.ShapeDtypeStruct((B * L,), jnp.float32),
              jax.ShapeDtypeStruct((B * L,), jnp.float32)),
    mesh=plsc.VectorSubcoreMesh(core_axis_name="c", subcore_axis_name="s"),
    scratch_types=[
        pltpu.VMEM((C,), jnp.int32),
        pltpu.VMEM((C,), jnp.int32),
        pltpu.VMEM((C * K,), jnp.int32),
        pltpu.VMEM((C * R, D), jnp.float32),
        pltpu.VMEM((BPW * L,), jnp.float32),
        pltpu.VMEM((BPW * L,), jnp.float32),
        pltpu.SemaphoreType.DMA,
    ],
)(_sc_body)


def _tc_loss_body(sp_ref, sn_ref, out_ref):
    sp = jnp.sum(sp_ref[...].reshape(B // 8, 8, L), axis=2)
    sn = jnp.sum(sn_ref[...].reshape(B // 8, 8, L), axis=2)

    def logsig(x):
        return jnp.minimum(x, 0.0) - jnp.log1p(jnp.exp(-jnp.abs(x)))

    out_ref[0, 0] = -(jnp.sum(logsig(sp)) + jnp.sum(logsig(-sn)))


_tc_loss = pl.pallas_call(
    _tc_loss_body,
    out_shape=jax.ShapeDtypeStruct((1, 1), jnp.float32),
    out_specs=pl.BlockSpec(memory_space=pltpu.SMEM),
)


def kernel(pos_v, pos_u, neg_u, v_table, u_table):
    pos_v = pos_v.astype(jnp.int32)
    pos_u = pos_u.astype(jnp.int32)
    neg_flat = neg_u.astype(jnp.int32).reshape(-1)
    sp, sn = _sc_dots(pos_v, pos_u, neg_flat, v_table, u_table)
    loss = _tc_loss(sp.reshape(B // 8, 128), sn.reshape(B // 8, 128))
    return loss[0, 0]
